# baseline (speedup 1.0000x reference)
import functools

import jax
import jax.numpy as jnp
from jax import lax
from jax.experimental import pallas as pl
from jax.experimental.pallas import tpu as pltpu
from jax.experimental.pallas import tpu_sc as plsc

_NUM_CODES = 8192
_CODE_DIM = 256
_BETA = 0.1
_BM = 256
_NBLK = 16384 // _BM


def _argmin_block(z_ref, e_ref, idx_ref):
    z = z_ref[...]
    emb = e_ref[...]
    z2 = jnp.sum(z * z, axis=1, keepdims=True)
    ones = jnp.ones((1, _CODE_DIM), jnp.float32)
    e2 = jax.lax.dot_general(
        ones, emb * emb, (((1,), (1,)), ((), ())))
    m = jax.lax.dot_general(
        z, emb, (((1,), (1,)), ((), ())))
    d = (z2 - 2.0 * m) + e2
    dmin = jnp.min(d, axis=1, keepdims=True)
    iota = jax.lax.broadcasted_iota(jnp.int32, d.shape, 1)
    idx = jnp.min(jnp.where(d == dmin, iota, _NUM_CODES), axis=1)
    idx_ref[...] = idx[None, None, :]


def _st_block(z_ref, zq_ref, out_ref, part_ref):
    z = z_ref[...]
    zq = zq_ref[...]
    out_ref[...] = z + (zq - z)
    part_ref[...] = jnp.sum((z - zq) ** 2)[None, None, None]


def _sc_gather(table_hbm, idx_hbm, out_hbm, idx_v, rows_v, sem):
    info = plsc.get_sparse_core_info()
    nc, ns = info.num_cores, info.num_subcores
    b_per_w = 16384 // (nc * ns)
    half = b_per_w // 2
    wid = lax.axis_index("s") * nc + lax.axis_index("c")
    base = wid * b_per_w
    for c in range(2):
        pltpu.sync_copy(idx_hbm.at[pl.ds(base + c * half, half)], idx_v)
        pltpu.async_copy(table_hbm.at[idx_v], rows_v, sem).wait()
        pltpu.sync_copy(rows_v, out_hbm.at[pl.ds(base + c * half, half)])


def kernel(z, embedding_weight):
    z_flat = z.reshape(-1, _CODE_DIM)

    idx_blocks = pl.pallas_call(
        _argmin_block,
        grid=(_NBLK,),
        in_specs=[
            pl.BlockSpec((_BM, _CODE_DIM), lambda i: (i, 0)),
            pl.BlockSpec((_NUM_CODES, _CODE_DIM), lambda i: (0, 0)),
        ],
        out_specs=pl.BlockSpec((1, 1, _BM), lambda i: (i, 0, 0)),
        out_shape=jax.ShapeDtypeStruct((_NBLK, 1, _BM), jnp.int32),
    )(z_flat, embedding_weight)
    idx = idx_blocks.reshape(-1)

    mesh = plsc.VectorSubcoreMesh(core_axis_name="c", subcore_axis_name="s")
    gather = functools.partial(
        pl.kernel,
        mesh=mesh,
        out_type=jax.ShapeDtypeStruct((16384, _CODE_DIM), jnp.float32),
        scratch_types=[
            pltpu.VMEM((256,), jnp.int32),
            pltpu.VMEM((256, _CODE_DIM), jnp.float32),
            pltpu.SemaphoreType.DMA,
        ],
    )(_sc_gather)
    zq = gather(embedding_weight, idx)

    zq_st, parts = pl.pallas_call(
        _st_block,
        grid=(_NBLK,),
        in_specs=[
            pl.BlockSpec((_BM, _CODE_DIM), lambda i: (i, 0)),
            pl.BlockSpec((_BM, _CODE_DIM), lambda i: (i, 0)),
        ],
        out_specs=[
            pl.BlockSpec((_BM, _CODE_DIM), lambda i: (i, 0)),
            pl.BlockSpec((1, 1, 1), lambda i: (i, 0, 0)),
        ],
        out_shape=[
            jax.ShapeDtypeStruct((16384, _CODE_DIM), jnp.float32),
            jax.ShapeDtypeStruct((_NBLK, 1, 1), jnp.float32),
        ],
    )(z_flat, zq)

    mean_sq = jnp.sum(parts) / (16384.0 * _CODE_DIM)
    vq_loss = _BETA * mean_sq + mean_sq
    return zq_st.reshape(z.shape), vq_loss
